# Initial kernel scaffold; baseline (speedup 1.0000x reference)
import jax
import jax.numpy as jnp
from jax.experimental import pallas as pl


def kernel(phases, S, G):
    raise NotImplementedError("write your pallas kernel here")


# speedup vs baseline: 1.1093x; 1.1093x over previous
import functools

import jax
import jax.numpy as jnp
from jax import lax
from jax.experimental import pallas as pl
from jax.experimental.pallas import tpu as pltpu
from jax.experimental.pallas import tpu_sc as plsc

N_NODES = 1024
P_PHASES = 12
BATCH = 64
EPS = 1e-06

NC = 2
NS = 16
NW = NC * NS
ROWS = N_NODES // NW
LANES = 16
NCH = N_NODES // LANES


def _gains_body(g_ref, out_ref):
    g = jax.nn.softplus(g_ref[...]) + 1e-06
    denom = jnp.maximum(jnp.sum(g, axis=-1, keepdims=True), EPS)
    out_ref[...] = g * (N_NODES / denom)


def _compute_gains(G):
    return pl.pallas_call(
        _gains_body,
        out_shape=jax.ShapeDtypeStruct((P_PHASES, N_NODES), jnp.float32),
    )(G)


def _sc_body(s_hbm, gains_hbm, order_hbm, starts_hbm, out_hbm,
             buf, gbuf, order_v, starts_v, st_sem):
    c = lax.axis_index("c")
    s = lax.axis_index("s")
    wid = s * NC + c
    base_row = pl.multiple_of(wid * ROWS, ROWS)
    lanes = lax.iota(jnp.int32, LANES)

    pltpu.sync_copy(order_hbm, order_v)
    pltpu.sync_copy(starts_hbm, starts_v)
    starts_vec = starts_v[...]

    for p in range(P_PHASES):
        slot = p % 2
        if p >= 2:
            prev = p - 2
            pstart = jnp.sum(jnp.where(lanes == prev, starts_vec, 0))
            pend = jnp.sum(jnp.where(lanes == prev + 1, starts_vec, 0))

            def drain_body(k, _):
                pltpu.make_async_copy(
                    s_hbm.at[0, pl.ds(0, ROWS)], buf.at[slot], st_sem
                ).wait()
                return 0

            lax.fori_loop(pstart, pend, drain_body, 0)

        pltpu.sync_copy(s_hbm.at[p, pl.ds(base_row, ROWS)], buf.at[slot])
        pltpu.sync_copy(gains_hbm.at[p, pl.ds(base_row, ROWS)], gbuf)

        def row_body(r, _):
            i_global = base_row + r

            def ch_body(ch, acc):
                off = pl.multiple_of(ch * LANES, LANES)
                x = buf[slot, r, pl.ds(off, LANES)]
                cols = off + lanes
                x = jnp.where(cols == i_global, 0.0, x)
                return acc + jnp.abs(x)

            acc = lax.fori_loop(0, NCH, ch_body, jnp.zeros((LANES,), jnp.float32))
            denom = jnp.maximum(jnp.sum(acc), EPS)
            gb = pl.multiple_of((r // LANES) * LANES, LANES)
            gchunk = gbuf[pl.ds(gb, LANES)]
            gval = jnp.sum(jnp.where(lanes == (r - gb), gchunk, 0.0))
            scale = jnp.full((LANES,), gval) / jnp.full((LANES,), denom)

            def ch2_body(ch, _):
                off = pl.multiple_of(ch * LANES, LANES)
                x = buf[slot, r, pl.ds(off, LANES)]
                cols = off + lanes
                buf[slot, r, pl.ds(off, LANES)] = jnp.where(
                    cols == i_global, 0.0, x * scale)
                return 0

            lax.fori_loop(0, NCH, ch2_body, 0)
            return 0

        lax.fori_loop(0, ROWS, row_body, 0)

        start = jnp.sum(jnp.where(lanes == p, starts_vec, 0))
        end = jnp.sum(jnp.where(lanes == (p + 1), starts_vec, 0))

        def w_body(k, _):
            kb = pl.multiple_of((k // LANES) * LANES, LANES)
            ochunk = order_v[pl.ds(kb, LANES)]
            b = jnp.sum(jnp.where(lanes == (k - kb), ochunk, 0))
            pltpu.async_copy(
                buf.at[slot], out_hbm.at[b, pl.ds(base_row, ROWS)], st_sem)
            return 0

        lax.fori_loop(start, end, w_body, 0)

    for p in (P_PHASES - 2, P_PHASES - 1):
        pstart = jnp.sum(jnp.where(lanes == p, starts_vec, 0))
        pend = jnp.sum(jnp.where(lanes == p + 1, starts_vec, 0))

        def tail_drain(k, _):
            pltpu.make_async_copy(
                s_hbm.at[0, pl.ds(0, ROWS)], buf.at[p % 2], st_sem
            ).wait()
            return 0

        lax.fori_loop(pstart, pend, tail_drain, 0)


_sc_call = functools.partial(
    pl.kernel,
    out_type=jax.ShapeDtypeStruct((BATCH, N_NODES, N_NODES), jnp.float32),
    mesh=plsc.VectorSubcoreMesh(
        core_axis_name="c", subcore_axis_name="s",
        num_cores=NC, num_subcores=NS),
    compiler_params=pltpu.CompilerParams(needs_layout_passes=False),
    scratch_types=[
        pltpu.VMEM((2, ROWS, N_NODES), jnp.float32),
        pltpu.VMEM((ROWS,), jnp.float32),
        pltpu.VMEM((BATCH,), jnp.int32),
        pltpu.VMEM((LANES,), jnp.int32),
        pltpu.SemaphoreType.DMA,
    ],
)(_sc_body)


@jax.jit
def kernel(phases, S, G):
    gains = _compute_gains(G)
    phases = phases.astype(jnp.int32)
    order = jnp.argsort(phases).astype(jnp.int32)
    sorted_ph = phases[order]
    starts = jnp.searchsorted(
        sorted_ph, jnp.arange(P_PHASES + 1, dtype=jnp.int32)).astype(jnp.int32)
    starts = jnp.pad(starts, (0, LANES - (P_PHASES + 1)),
                     constant_values=BATCH)
    return _sc_call(S, gains, order, starts)


# speedup vs baseline: 2.3087x; 2.0812x over previous
import functools

import jax
import jax.numpy as jnp
from jax import lax
from jax.experimental import pallas as pl
from jax.experimental.pallas import tpu as pltpu
from jax.experimental.pallas import tpu_sc as plsc

N_NODES = 1024
P_PHASES = 12
BATCH = 64
EPS = 1e-06

NC = 2
NS = 16
NW = NC * NS
ROWS = N_NODES // NW
LANES = 16
NCH = N_NODES // LANES


def _gains_body(g_ref, out_ref):
    g = jax.nn.softplus(g_ref[...]) + 1e-06
    denom = jnp.maximum(jnp.sum(g, axis=-1, keepdims=True), EPS)
    out_ref[...] = g * (N_NODES / denom)


def _compute_gains(G):
    return pl.pallas_call(
        _gains_body,
        out_shape=jax.ShapeDtypeStruct((P_PHASES, N_NODES), jnp.float32),
    )(G)


def _sc_body(s_hbm, gains_hbm, order_hbm, starts_hbm, out_hbm,
             buf, gbuf, order_v, starts_v, sem0, sem1):
    c = lax.axis_index("c")
    s = lax.axis_index("s")
    wid = s * NC + c
    base_row = pl.multiple_of(wid * ROWS, ROWS)
    lanes = lax.iota(jnp.int32, LANES)

    pltpu.sync_copy(order_hbm, order_v)
    pltpu.sync_copy(starts_hbm, starts_v)
    starts_vec = starts_v[...]
    sems = (sem0, sem1)

    def sel_i32(vec, i):
        return jnp.sum(jnp.where(lanes == i, vec, 0))

    def phase_work(p, slot):
        dstart = sel_i32(starts_vec, p - 2)
        dend = sel_i32(starts_vec, p - 1)

        def drain_body(k, _):
            pltpu.make_async_copy(
                s_hbm.at[0, pl.ds(0, ROWS)], buf.at[slot], sems[slot]).wait()
            return 0

        lax.fori_loop(dstart, dend, drain_body, 0)

        pltpu.sync_copy(s_hbm.at[p, pl.ds(base_row, ROWS)], buf.at[slot])
        pltpu.sync_copy(gains_hbm.at[p, pl.ds(base_row, ROWS)], gbuf)

        def row_body(r, _):
            i_global = base_row + r
            accs = [jnp.zeros((LANES,), jnp.float32) for _ in range(4)]
            for ch in range(NCH):
                x = buf[slot, r, pl.ds(ch * LANES, LANES)]
                accs[ch % 4] = accs[ch % 4] + jnp.abs(x)
            dch = pl.multiple_of((i_global // LANES) * LANES, LANES)
            dlane = i_global - dch
            dchunk = buf[slot, r, pl.ds(dch, LANES)]
            acc = (accs[0] + accs[1]) + (accs[2] + accs[3])
            acc = acc - jnp.where(lanes == dlane, jnp.abs(dchunk), 0.0)
            denom = jnp.maximum(jnp.sum(acc), EPS)
            gb = pl.multiple_of((r // LANES) * LANES, LANES)
            gchunk = gbuf[pl.ds(gb, LANES)]
            gval = jnp.sum(jnp.where(lanes == (r - gb), gchunk, 0.0))
            scale = jnp.full((LANES,), gval) / jnp.full((LANES,), denom)
            for ch in range(NCH):
                ds = pl.ds(ch * LANES, LANES)
                buf[slot, r, ds] = buf[slot, r, ds] * scale
            dchunk2 = buf[slot, r, pl.ds(dch, LANES)]
            buf[slot, r, pl.ds(dch, LANES)] = jnp.where(
                lanes == dlane, 0.0, dchunk2)
            return 0

        lax.fori_loop(0, ROWS, row_body, 0)

        start = sel_i32(starts_vec, p)
        end = sel_i32(starts_vec, p + 1)

        def w_body(k, _):
            kb = pl.multiple_of((k // LANES) * LANES, LANES)
            ochunk = order_v[pl.ds(kb, LANES)]
            b = sel_i32(ochunk, k - kb)
            pltpu.async_copy(
                buf.at[slot], out_hbm.at[b, pl.ds(base_row, ROWS)], sems[slot])
            return 0

        lax.fori_loop(start, end, w_body, 0)

    def pair_body(t, _):
        phase_work(2 * t, 0)
        phase_work(2 * t + 1, 1)
        return 0

    lax.fori_loop(0, P_PHASES // 2, pair_body, 0)

    for p in (P_PHASES - 2, P_PHASES - 1):
        pstart = sel_i32(starts_vec, p)
        pend = sel_i32(starts_vec, p + 1)
        slot = p % 2

        def tail_drain(k, _):
            pltpu.make_async_copy(
                s_hbm.at[0, pl.ds(0, ROWS)], buf.at[slot], sems[slot]).wait()
            return 0

        lax.fori_loop(pstart, pend, tail_drain, 0)


_sc_call = functools.partial(
    pl.kernel,
    out_type=jax.ShapeDtypeStruct((BATCH, N_NODES, N_NODES), jnp.float32),
    mesh=plsc.VectorSubcoreMesh(
        core_axis_name="c", subcore_axis_name="s",
        num_cores=NC, num_subcores=NS),
    compiler_params=pltpu.CompilerParams(needs_layout_passes=False),
    scratch_types=[
        pltpu.VMEM((2, ROWS, N_NODES), jnp.float32),
        pltpu.VMEM((ROWS,), jnp.float32),
        pltpu.VMEM((BATCH,), jnp.int32),
        pltpu.VMEM((LANES,), jnp.int32),
        pltpu.SemaphoreType.DMA,
        pltpu.SemaphoreType.DMA,
    ],
)(_sc_body)


@jax.jit
def kernel(phases, S, G):
    gains = _compute_gains(G)
    phases = phases.astype(jnp.int32)
    order = jnp.argsort(phases).astype(jnp.int32)
    sorted_ph = phases[order]
    starts = jnp.searchsorted(
        sorted_ph, jnp.arange(P_PHASES + 1, dtype=jnp.int32)).astype(jnp.int32)
    starts = jnp.pad(starts, (0, LANES - (P_PHASES + 1)),
                     constant_values=BATCH)
    return _sc_call(S, gains, order, starts)


# speedup vs baseline: 2.4736x; 1.0714x over previous
import functools

import jax
import jax.numpy as jnp
from jax import lax
from jax.experimental import pallas as pl
from jax.experimental.pallas import tpu as pltpu
from jax.experimental.pallas import tpu_sc as plsc

N_NODES = 1024
P_PHASES = 12
BATCH = 64
EPS = 1e-06

NC = 2
NS = 16
NW = NC * NS
ROWS = N_NODES // NW
LANES = 16
NCH = N_NODES // LANES


def _gains_body(g_ref, out_ref):
    g = jax.nn.softplus(g_ref[...]) + 1e-06
    denom = jnp.maximum(jnp.sum(g, axis=-1, keepdims=True), EPS)
    out_ref[...] = g * (N_NODES / denom)


def _compute_gains(G):
    return pl.pallas_call(
        _gains_body,
        out_shape=jax.ShapeDtypeStruct((P_PHASES, N_NODES), jnp.float32),
    )(G)


def _sc_body(s_hbm, gains_hbm, order_hbm, starts_hbm, out_hbm,
             buf, gbuf, order_v, starts_v, sem0, sem1, sem2, rd_sem):
    c = lax.axis_index("c")
    s = lax.axis_index("s")
    wid = s * NC + c
    base_row = pl.multiple_of(wid * ROWS, ROWS)
    lanes = lax.iota(jnp.int32, LANES)

    pltpu.sync_copy(order_hbm, order_v)
    pltpu.sync_copy(starts_hbm, starts_v)
    starts_vec = starts_v[...]
    sems = (sem0, sem1, sem2)

    def sel_i32(vec, i):
        return jnp.sum(jnp.where(lanes == i, vec, 0))

    pltpu.async_copy(s_hbm.at[0, pl.ds(base_row, ROWS)], buf.at[0], rd_sem)

    def phase_work(p, slot, nslot):
        dstart = sel_i32(starts_vec, p - 2)
        dend = sel_i32(starts_vec, p - 1)

        def drain_body(k, _):
            pltpu.make_async_copy(
                s_hbm.at[0, pl.ds(0, ROWS)], buf.at[nslot], sems[nslot]).wait()
            return 0

        lax.fori_loop(dstart, dend, drain_body, 0)

        pltpu.make_async_copy(
            s_hbm.at[0, pl.ds(0, ROWS)], buf.at[slot], rd_sem).wait()
        pnext = jnp.minimum(p + 1, P_PHASES - 1)
        pltpu.async_copy(
            s_hbm.at[pnext, pl.ds(base_row, ROWS)], buf.at[nslot], rd_sem)
        pltpu.sync_copy(gains_hbm.at[p, pl.ds(base_row, ROWS)], gbuf)

        def row_body(r, _):
            i_global = base_row + r
            accs = [jnp.zeros((LANES,), jnp.float32) for _ in range(4)]
            for ch in range(NCH):
                x = buf[slot, r, pl.ds(ch * LANES, LANES)]
                accs[ch % 4] = accs[ch % 4] + jnp.abs(x)
            dch = pl.multiple_of((i_global // LANES) * LANES, LANES)
            dlane = i_global - dch
            dchunk = buf[slot, r, pl.ds(dch, LANES)]
            acc = (accs[0] + accs[1]) + (accs[2] + accs[3])
            acc = acc - jnp.where(lanes == dlane, jnp.abs(dchunk), 0.0)
            denom = jnp.maximum(jnp.sum(acc), EPS)
            gb = pl.multiple_of((r // LANES) * LANES, LANES)
            gchunk = gbuf[pl.ds(gb, LANES)]
            gval = jnp.sum(jnp.where(lanes == (r - gb), gchunk, 0.0))
            scale = jnp.full((LANES,), gval) / jnp.full((LANES,), denom)
            for ch in range(NCH):
                ds = pl.ds(ch * LANES, LANES)
                buf[slot, r, ds] = buf[slot, r, ds] * scale
            dchunk2 = buf[slot, r, pl.ds(dch, LANES)]
            buf[slot, r, pl.ds(dch, LANES)] = jnp.where(
                lanes == dlane, 0.0, dchunk2)
            return 0

        lax.fori_loop(0, ROWS, row_body, 0)

        start = sel_i32(starts_vec, p)
        end = sel_i32(starts_vec, p + 1)

        def w_body(k, _):
            kb = pl.multiple_of((k // LANES) * LANES, LANES)
            ochunk = order_v[pl.ds(kb, LANES)]
            b = sel_i32(ochunk, k - kb)
            pltpu.async_copy(
                buf.at[slot], out_hbm.at[b, pl.ds(base_row, ROWS)], sems[slot])
            return 0

        lax.fori_loop(start, end, w_body, 0)

    def triple_body(t, _):
        phase_work(3 * t, 0, 1)
        phase_work(3 * t + 1, 1, 2)
        phase_work(3 * t + 2, 2, 0)
        return 0

    lax.fori_loop(0, P_PHASES // 3, triple_body, 0)

    pltpu.make_async_copy(
        s_hbm.at[0, pl.ds(0, ROWS)], buf.at[0], rd_sem).wait()

    for p in (P_PHASES - 2, P_PHASES - 1):
        pstart = sel_i32(starts_vec, p)
        pend = sel_i32(starts_vec, p + 1)
        slot = p % 3

        def tail_drain(k, _):
            pltpu.make_async_copy(
                s_hbm.at[0, pl.ds(0, ROWS)], buf.at[slot], sems[slot]).wait()
            return 0

        lax.fori_loop(pstart, pend, tail_drain, 0)


_sc_call = functools.partial(
    pl.kernel,
    out_type=jax.ShapeDtypeStruct((BATCH, N_NODES, N_NODES), jnp.float32),
    mesh=plsc.VectorSubcoreMesh(
        core_axis_name="c", subcore_axis_name="s",
        num_cores=NC, num_subcores=NS),
    compiler_params=pltpu.CompilerParams(needs_layout_passes=False),
    scratch_types=[
        pltpu.VMEM((3, ROWS, N_NODES), jnp.float32),
        pltpu.VMEM((ROWS,), jnp.float32),
        pltpu.VMEM((BATCH,), jnp.int32),
        pltpu.VMEM((LANES,), jnp.int32),
        pltpu.SemaphoreType.DMA,
        pltpu.SemaphoreType.DMA,
        pltpu.SemaphoreType.DMA,
        pltpu.SemaphoreType.DMA,
    ],
)(_sc_body)


@jax.jit
def kernel(phases, S, G):
    gains = _compute_gains(G)
    phases = phases.astype(jnp.int32)
    order = jnp.argsort(phases).astype(jnp.int32)
    sorted_ph = phases[order]
    starts = jnp.searchsorted(
        sorted_ph, jnp.arange(P_PHASES + 1, dtype=jnp.int32)).astype(jnp.int32)
    starts = jnp.pad(starts, (0, LANES - (P_PHASES + 1)),
                     constant_values=BATCH)
    return _sc_call(S, gains, order, starts)


# speedup vs baseline: 2.5373x; 1.0257x over previous
import functools

import jax
import jax.numpy as jnp
from jax import lax
from jax.experimental import pallas as pl
from jax.experimental.pallas import tpu as pltpu
from jax.experimental.pallas import tpu_sc as plsc

N_NODES = 1024
P_PHASES = 12
BATCH = 64
EPS = 1e-06

NC = 2
NS = 16
NW = NC * NS
ROWS = N_NODES // NW
LANES = 16
NCH = N_NODES // LANES


def _prep_body(ph_ref, g_ref, gains_ref, order_ref, starts_ref):
    g = jax.nn.softplus(g_ref[...]) + 1e-06
    denom = jnp.maximum(jnp.sum(g, axis=-1, keepdims=True), EPS)
    gains_ref[...] = g * (N_NODES / denom)

    ph = ph_ref[...]
    prow = lax.broadcasted_iota(jnp.int32, (LANES, BATCH), 0)
    ohp = (ph == prow).astype(jnp.float32)
    counts = jnp.sum(ohp, axis=1, keepdims=True)
    lrow = lax.broadcasted_iota(jnp.int32, (LANES, LANES), 0)
    lcol = lax.broadcasted_iota(jnp.int32, (LANES, LANES), 1)
    ls = (lcol < lrow).astype(jnp.float32)
    starts = jnp.dot(ls, counts, preferred_element_type=jnp.float32)
    brow = lax.broadcasted_iota(jnp.int32, (BATCH, BATCH), 0)
    bcol = lax.broadcasted_iota(jnp.int32, (BATCH, BATCH), 1)
    ub = (brow < bcol).astype(jnp.float32)
    pre = jnp.dot(ohp, ub, preferred_element_type=jnp.float32)
    rank = jnp.sum(pre * ohp, axis=0, keepdims=True)
    startsb = jnp.sum(starts * ohp, axis=0, keepdims=True)
    pos = (rank + startsb).astype(jnp.int32)
    eq = (brow == pos)
    order = jnp.sum(
        eq.astype(jnp.float32) * bcol.astype(jnp.float32), axis=1,
        keepdims=True)
    order_ref[...] = order.astype(jnp.int32)
    starts_ref[...] = starts.astype(jnp.int32)


def _prep(phases, G):
    return pl.pallas_call(
        _prep_body,
        out_shape=(
            jax.ShapeDtypeStruct((P_PHASES, N_NODES), jnp.float32),
            jax.ShapeDtypeStruct((BATCH, 1), jnp.int32),
            jax.ShapeDtypeStruct((LANES, 1), jnp.int32),
        ),
    )(phases.reshape(1, BATCH), G)


def _sc_body(s_hbm, gains_hbm, order_hbm, starts_hbm, out_hbm,
             buf, gbuf, order_v, starts_v, sem0, sem1, sem2, rd_sem):
    c = lax.axis_index("c")
    s = lax.axis_index("s")
    wid = s * NC + c
    base_row = pl.multiple_of(wid * ROWS, ROWS)
    lanes = lax.iota(jnp.int32, LANES)

    pltpu.sync_copy(order_hbm, order_v)
    pltpu.sync_copy(starts_hbm, starts_v)
    starts_vec = starts_v[...]
    sems = (sem0, sem1, sem2)

    def sel_i32(vec, i):
        return jnp.sum(jnp.where(lanes == i, vec, 0))

    pltpu.async_copy(s_hbm.at[0, pl.ds(base_row, ROWS)], buf.at[0], rd_sem)

    def phase_work(p, slot, nslot):
        dstart = sel_i32(starts_vec, p - 2)
        dend = sel_i32(starts_vec, p - 1)

        def drain_body(k, _):
            pltpu.make_async_copy(
                s_hbm.at[0, pl.ds(0, ROWS)], buf.at[nslot], sems[nslot]).wait()
            return 0

        lax.fori_loop(dstart, dend, drain_body, 0)

        pltpu.make_async_copy(
            s_hbm.at[0, pl.ds(0, ROWS)], buf.at[slot], rd_sem).wait()
        pnext = jnp.minimum(p + 1, P_PHASES - 1)
        pltpu.async_copy(
            s_hbm.at[pnext, pl.ds(base_row, ROWS)], buf.at[nslot], rd_sem)
        pltpu.sync_copy(gains_hbm.at[p, pl.ds(base_row, ROWS)], gbuf)

        def row_body(r, _):
            i_global = base_row + r
            accs = [jnp.zeros((LANES,), jnp.float32) for _ in range(4)]
            for ch in range(NCH):
                x = buf[slot, r, pl.ds(ch * LANES, LANES)]
                accs[ch % 4] = accs[ch % 4] + jnp.abs(x)
            dch = pl.multiple_of((i_global // LANES) * LANES, LANES)
            dlane = i_global - dch
            dchunk = buf[slot, r, pl.ds(dch, LANES)]
            acc = (accs[0] + accs[1]) + (accs[2] + accs[3])
            acc = acc - jnp.where(lanes == dlane, jnp.abs(dchunk), 0.0)
            denom = jnp.maximum(jnp.sum(acc), EPS)
            gb = pl.multiple_of((r // LANES) * LANES, LANES)
            gchunk = gbuf[pl.ds(gb, LANES)]
            gval = jnp.sum(jnp.where(lanes == (r - gb), gchunk, 0.0))
            scale = jnp.full((LANES,), gval) / jnp.full((LANES,), denom)
            for ch in range(NCH):
                ds = pl.ds(ch * LANES, LANES)
                buf[slot, r, ds] = buf[slot, r, ds] * scale
            dchunk2 = buf[slot, r, pl.ds(dch, LANES)]
            buf[slot, r, pl.ds(dch, LANES)] = jnp.where(
                lanes == dlane, 0.0, dchunk2)
            return 0

        lax.fori_loop(0, ROWS, row_body, 0)

        start = sel_i32(starts_vec, p)
        end = sel_i32(starts_vec, p + 1)

        def w_body(k, _):
            kb = pl.multiple_of((k // LANES) * LANES, LANES)
            ochunk = order_v[pl.ds(kb, LANES)]
            b = sel_i32(ochunk, k - kb)
            pltpu.async_copy(
                buf.at[slot], out_hbm.at[b, pl.ds(base_row, ROWS)], sems[slot])
            return 0

        lax.fori_loop(start, end, w_body, 0)

    def triple_body(t, _):
        phase_work(3 * t, 0, 1)
        phase_work(3 * t + 1, 1, 2)
        phase_work(3 * t + 2, 2, 0)
        return 0

    lax.fori_loop(0, P_PHASES // 3, triple_body, 0)

    pltpu.make_async_copy(
        s_hbm.at[0, pl.ds(0, ROWS)], buf.at[0], rd_sem).wait()

    for p in (P_PHASES - 2, P_PHASES - 1):
        pstart = sel_i32(starts_vec, p)
        pend = sel_i32(starts_vec, p + 1)
        slot = p % 3

        def tail_drain(k, _):
            pltpu.make_async_copy(
                s_hbm.at[0, pl.ds(0, ROWS)], buf.at[slot], sems[slot]).wait()
            return 0

        lax.fori_loop(pstart, pend, tail_drain, 0)


_sc_call = functools.partial(
    pl.kernel,
    out_type=jax.ShapeDtypeStruct((BATCH, N_NODES, N_NODES), jnp.float32),
    mesh=plsc.VectorSubcoreMesh(
        core_axis_name="c", subcore_axis_name="s",
        num_cores=NC, num_subcores=NS),
    compiler_params=pltpu.CompilerParams(needs_layout_passes=False),
    scratch_types=[
        pltpu.VMEM((3, ROWS, N_NODES), jnp.float32),
        pltpu.VMEM((ROWS,), jnp.float32),
        pltpu.VMEM((BATCH,), jnp.int32),
        pltpu.VMEM((LANES,), jnp.int32),
        pltpu.SemaphoreType.DMA,
        pltpu.SemaphoreType.DMA,
        pltpu.SemaphoreType.DMA,
        pltpu.SemaphoreType.DMA,
    ],
)(_sc_body)


@jax.jit
def kernel(phases, S, G):
    gains, order, starts = _prep(phases.astype(jnp.int32), G)
    return _sc_call(S, gains, order.reshape(BATCH), starts.reshape(LANES))
